# Initial kernel scaffold; baseline (speedup 1.0000x reference)
import functools

import jax
import jax.numpy as jnp
from jax import lax
from jax.experimental import pallas as pl
from jax.experimental.pallas import tpu as pltpu
from jax.experimental.pallas import tpu_sc as plsc

NUM_WORKERS = 32
BATCH = 16384
SEQ = 200
D = 64
ROWS_PER_W = BATCH // NUM_WORKERS
CHUNK = 4
IDX_PER_CHUNK = CHUNK * SEQ
N_CHUNKS = ROWS_PER_W // CHUNK
GSLICE = 80
N_GATHERS = IDX_PER_CHUNK // GSLICE


def _sc_body(labels_hbm, table_hbm, out_hbm,
             idx_e, idx_o, rows_e, rows_o, out_stage, sem_e, sem_o):
    wid = lax.axis_index("s") * 2 + lax.axis_index("c")
    w_idx_base = wid * ROWS_PER_W * SEQ
    w_row_base = wid * ROWS_PER_W

    def load_idx(chunk, idx_v):
        pltpu.sync_copy(
            labels_hbm.at[pl.ds(w_idx_base + chunk * IDX_PER_CHUNK,
                                IDX_PER_CHUNK)],
            idx_v)

    def fire_gathers(idx_v, rows_v, sem):
        for j in range(N_GATHERS):
            pltpu.async_copy(
                table_hbm.at[idx_v.at[pl.ds(j * GSLICE, GSLICE)]],
                rows_v.at[pl.ds(j * GSLICE, GSLICE)],
                sem)

    def drain(idx_v, rows_v, sem):
        for j in range(N_GATHERS):
            pltpu.make_async_copy(
                table_hbm.at[idx_v.at[pl.ds(j * GSLICE, GSLICE)]],
                rows_v.at[pl.ds(j * GSLICE, GSLICE)],
                sem).wait()

    inv = jnp.float32(1.0 / SEQ)

    def accumulate(rows_v):
        for i in range(CHUNK):
            base = i * SEQ

            def body(j, acc):
                r = base + 4 * j
                a0, a1, a2, a3 = acc
                for k in range(4):
                    a0 = a0 + rows_v[r + k, pl.ds(0, 16)]
                    a1 = a1 + rows_v[r + k, pl.ds(16, 16)]
                    a2 = a2 + rows_v[r + k, pl.ds(32, 16)]
                    a3 = a3 + rows_v[r + k, pl.ds(48, 16)]
                return (a0, a1, a2, a3)

            z = jnp.zeros((16,), jnp.float32)
            a0, a1, a2, a3 = lax.fori_loop(0, SEQ // 4, body, (z, z, z, z))
            out_stage[i, pl.ds(0, 16)] = a0 * inv
            out_stage[i, pl.ds(16, 16)] = a1 * inv
            out_stage[i, pl.ds(32, 16)] = a2 * inv
            out_stage[i, pl.ds(48, 16)] = a3 * inv

    def store_out(chunk):
        pltpu.sync_copy(out_stage,
                        out_hbm.at[pl.ds(w_row_base + chunk * CHUNK, CHUNK)])

    load_idx(0, idx_e)
    fire_gathers(idx_e, rows_e, sem_e)

    def outer(g0, carry):
        c0 = 2 * g0
        c1 = 2 * g0 + 1

        load_idx(c1, idx_o)
        fire_gathers(idx_o, rows_o, sem_o)

        drain(idx_e, rows_e, sem_e)
        accumulate(rows_e)
        store_out(c0)

        @pl.when(g0 < N_CHUNKS // 2 - 1)
        def _():
            load_idx(c0 + 2, idx_e)
            fire_gathers(idx_e, rows_e, sem_e)

        drain(idx_o, rows_o, sem_o)
        accumulate(rows_o)
        store_out(c1)
        return carry

    lax.fori_loop(0, N_CHUNKS // 2, outer, 0)


def kernel(labels, table):
    labels_flat = labels.reshape(BATCH * SEQ).astype(jnp.int32)
    mesh = plsc.VectorSubcoreMesh(core_axis_name="c", subcore_axis_name="s")
    f = pl.kernel(
        _sc_body,
        out_type=jax.ShapeDtypeStruct((BATCH, D), jnp.float32),
        mesh=mesh,
        scratch_types=[
            pltpu.VMEM((IDX_PER_CHUNK,), jnp.int32),
            pltpu.VMEM((IDX_PER_CHUNK,), jnp.int32),
            pltpu.VMEM((IDX_PER_CHUNK, D), jnp.float32),
            pltpu.VMEM((IDX_PER_CHUNK, D), jnp.float32),
            pltpu.VMEM((CHUNK, D), jnp.float32),
            pltpu.SemaphoreType.DMA,
            pltpu.SemaphoreType.DMA,
        ],
    )
    return f(labels_flat, table)


# speedup vs baseline: 3.2219x; 3.2219x over previous
import functools

import jax
import jax.numpy as jnp
from jax import lax
from jax.experimental import pallas as pl
from jax.experimental.pallas import tpu as pltpu
from jax.experimental.pallas import tpu_sc as plsc

NUM_WORKERS = 32
BATCH = 16384
SEQ = 200
D = 64
ROWS_PER_W = BATCH // NUM_WORKERS
CHUNK = 4
IDX_PER_CHUNK = CHUNK * SEQ
N_CHUNKS = ROWS_PER_W // CHUNK
GSLICE = 80
N_GATHERS = IDX_PER_CHUNK // GSLICE


def _sc_body(labels_hbm, table_hbm, out_hbm,
             idx_e, idx_o, rows_e, rows_o, out_stage, sem_e, sem_o):
    wid = lax.axis_index("s") * 2 + lax.axis_index("c")
    w_idx_base = wid * ROWS_PER_W * SEQ
    w_row_base = wid * ROWS_PER_W

    def load_idx(chunk, idx_v):
        pltpu.sync_copy(
            labels_hbm.at[pl.ds(w_idx_base + chunk * IDX_PER_CHUNK,
                                IDX_PER_CHUNK)],
            idx_v)

    def fire_gathers(idx_v, rows_v, sem):
        for j in range(N_GATHERS):
            pltpu.async_copy(
                table_hbm.at[idx_v.at[pl.ds(j * GSLICE, GSLICE)]],
                rows_v.at[pl.ds(j * GSLICE, GSLICE)],
                sem)

    def drain(idx_v, rows_v, sem):
        for j in range(N_GATHERS):
            pltpu.make_async_copy(
                table_hbm.at[idx_v.at[pl.ds(j * GSLICE, GSLICE)]],
                rows_v.at[pl.ds(j * GSLICE, GSLICE)],
                sem).wait()

    inv = jnp.float32(1.0 / SEQ)

    def accumulate(rows_v):
        for i in range(CHUNK):
            base = i * SEQ

            def body(j, acc):
                r = base + 4 * j
                a0, a1, a2, a3 = acc
                for k in range(4):
                    a0 = a0 + rows_v[r + k, pl.ds(0, 16)]
                    a1 = a1 + rows_v[r + k, pl.ds(16, 16)]
                    a2 = a2 + rows_v[r + k, pl.ds(32, 16)]
                    a3 = a3 + rows_v[r + k, pl.ds(48, 16)]
                return (a0, a1, a2, a3)

            z = jnp.zeros((16,), jnp.float32)
            a0, a1, a2, a3 = lax.fori_loop(0, SEQ // 4, body, (z, z, z, z))
            out_stage[i, pl.ds(0, 16)] = a0 * inv
            out_stage[i, pl.ds(16, 16)] = a1 * inv
            out_stage[i, pl.ds(32, 16)] = a2 * inv
            out_stage[i, pl.ds(48, 16)] = a3 * inv

    def store_out(chunk):
        pltpu.sync_copy(out_stage,
                        out_hbm.at[pl.ds(w_row_base + chunk * CHUNK, CHUNK)])

    load_idx(0, idx_e)
    fire_gathers(idx_e, rows_e, sem_e)

    def outer(g0, carry):
        c0 = 2 * g0
        c1 = 2 * g0 + 1

        load_idx(c1, idx_o)
        fire_gathers(idx_o, rows_o, sem_o)

        drain(idx_e, rows_e, sem_e)
        accumulate(rows_e)
        store_out(c0)

        @pl.when(g0 < N_CHUNKS // 2 - 1)
        def _():
            load_idx(c0 + 2, idx_e)
            fire_gathers(idx_e, rows_e, sem_e)

        drain(idx_o, rows_o, sem_o)
        accumulate(rows_o)
        store_out(c1)
        return carry

    lax.fori_loop(0, N_CHUNKS // 2, outer, 0)


def kernel(labels, table):
    labels_flat = labels.reshape(BATCH * SEQ).astype(jnp.int32)
    mesh = plsc.VectorSubcoreMesh(core_axis_name="c", subcore_axis_name="s")
    f = pl.kernel(
        _sc_body,
        out_type=jax.ShapeDtypeStruct((BATCH, D), jnp.float32),
        mesh=mesh,
        scratch_types=[
            pltpu.VMEM((IDX_PER_CHUNK,), jnp.int32),
            pltpu.VMEM((IDX_PER_CHUNK,), jnp.int32),
            pltpu.VMEM((IDX_PER_CHUNK, D), jnp.float32),
            pltpu.VMEM((IDX_PER_CHUNK, D), jnp.float32),
            pltpu.VMEM((CHUNK, D), jnp.float32),
            pltpu.SemaphoreType.DMA,
            pltpu.SemaphoreType.DMA,
        ],
        compiler_params=pltpu.CompilerParams(use_tc_tiling_on_sc=False),
    )
    return f(labels_flat, table)
